# baseline (speedup 1.0000x reference)
import jax
import jax.numpy as jnp
import numpy as np
from jax import lax
from jax.experimental import pallas as pl
from jax.experimental.pallas import tpu as pltpu

DIMS = 3
T = 8192
N_LM = 543
LIPS = ([61, 185, 40, 39, 37, 0, 267, 269, 270, 409, 291]
        + [146, 91, 181, 84, 17, 314, 405, 321, 375, 291]
        + [78, 191, 80, 81, 82, 13, 312, 311, 310, 415, 308]
        + [78, 95, 88, 178, 87, 14, 317, 402, 318, 324, 308])
HL_LO, HL_HI = 468, 489
POSE_LO, POSE_HI = 489, 522
HR_LO, HR_HI = 522, 543

_LIP_TILES = sorted({lm // 8 for lm in LIPS})
N_LT = len(_LIP_TILES)
FB = 1024
NF = T // FB

N_LIPS = len(LIPS)
_FEAT_LM = np.asarray(
    LIPS + list(range(HL_LO, HL_HI)) + list(range(POSE_LO, POSE_HI))
    + list(range(HR_LO, HR_HI)), np.int32)
N_FD = _FEAT_LM.shape[0]

_GT_np = np.zeros((N_LT, 8, N_FD), np.float32)
for _j, _lm in enumerate(LIPS):
    _GT_np[_LIP_TILES.index(_lm // 8), _lm % 8, _j] = 1.0

_GB_np = np.zeros((128, N_FD), np.float32)
for _j, _lm in enumerate(_FEAT_LM[N_LIPS:], start=N_LIPS):
    _GB_np[int(_lm) - 448, _j] = 1.0

F_HL_LO, F_HL_HI = N_LIPS, N_LIPS + 21
F_HR_LO, F_HR_HI = N_FD - 21, N_FD


def _lips_kernel(tbl_ref, lip_ref, gt_ref, out_ref, acc8s, acc8q, fs, fq):
    j = pl.program_id(0)
    f = pl.program_id(1)

    @pl.when(jnp.logical_and(j == 0, f == 0))
    def _init():
        fs[...] = jnp.zeros_like(fs)
        fq[...] = jnp.zeros_like(fq)

    @pl.when(f == 0)
    def _reset():
        acc8s[...] = jnp.zeros_like(acc8s)
        acc8q[...] = jnp.zeros_like(acc8q)

    lb = lip_ref[...]
    acc8s[...] += jnp.sum(lb, axis=2)
    acc8q[...] += jnp.sum(lb * lb, axis=2)

    @pl.when(f == NF - 1)
    def _flush():
        gt = gt_ref[0]
        fs[...] += jnp.dot(acc8s[...], gt,
                           preferred_element_type=jnp.float32,
                           precision=jax.lax.Precision.HIGHEST)
        fq[...] += jnp.dot(acc8q[...], gt,
                           preferred_element_type=jnp.float32,
                           precision=jax.lax.Precision.HIGHEST)

    @pl.when(jnp.logical_and(j == N_LT - 1, f == NF - 1))
    def _emit():
        out_ref[0:DIMS, :] = fs[...]
        out_ref[DIMS:, :] = fq[...]


def _tail_kernel(ha_ref, hb_ref, lips_ref, gb_ref, out_ref,
                 acc_s, acc_q, acc_n):
    f = pl.program_id(0)

    @pl.when(f == 0)
    def _init():
        acc_s[...] = jnp.zeros_like(acc_s)
        acc_q[...] = jnp.zeros_like(acc_q)
        acc_n[0] = 0.0
        acc_n[1] = 0.0

    ba = ha_ref[...]
    bb = hb_ref[...]
    hl = ba[:, HL_LO - 448:HL_HI - 448, :]
    hr = bb[:, HR_LO - 512:HR_HI - 512, :]
    hl_bad = jnp.any(jnp.any(jnp.isnan(hl), axis=1, keepdims=True),
                     axis=0, keepdims=True).astype(jnp.float32)
    hr_bad = jnp.any(jnp.any(jnp.isnan(hr), axis=1, keepdims=True),
                     axis=0, keepdims=True).astype(jnp.float32)

    lma = lax.broadcasted_iota(jnp.int32, (1, 64, 1), 1)
    is_hl = jnp.logical_and(lma >= HL_LO - 448,
                            lma < HL_HI - 448).astype(jnp.float32)
    is_hr = jnp.logical_and(lma >= HR_LO - 512,
                            lma < HR_HI - 512).astype(jnp.float32)
    is_pad = (lma >= N_LM - 512).astype(jnp.float32)
    wa = jnp.broadcast_to(hl_bad * is_hl, (DIMS, 64, FB))
    wb = jnp.broadcast_to(hr_bad * is_hr + is_pad, (DIMS, 64, FB))
    ba2 = jnp.where(wa == 0.0, ba, 0.0)
    bb2 = jnp.where(wb == 0.0, bb, 0.0)

    acc_s[:, 0:64] += jnp.sum(ba2, axis=2)
    acc_q[:, 0:64] += jnp.sum(ba2 * ba2, axis=2)
    acc_s[:, 64:128] += jnp.sum(bb2, axis=2)
    acc_q[:, 64:128] += jnp.sum(bb2 * bb2, axis=2)
    acc_n[0] += jnp.float32(FB) - jnp.sum(hl_bad)
    acc_n[1] += jnp.float32(FB) - jnp.sum(hr_bad)

    @pl.when(f == NF - 1)
    def _finalize():
        gb = gb_ref[...]
        s = lips_ref[0:DIMS, :] + jnp.dot(
            acc_s[...], gb, preferred_element_type=jnp.float32,
            precision=jax.lax.Precision.HIGHEST)
        q = lips_ref[DIMS:, :] + jnp.dot(
            acc_q[...], gb, preferred_element_type=jnp.float32,
            precision=jax.lax.Precision.HIGHEST)
        col = lax.broadcasted_iota(jnp.int32, (DIMS, N_FD), 1)
        in_hl = jnp.logical_and(col >= F_HL_LO,
                                col < F_HL_HI).astype(jnp.float32)
        in_hr = (col >= F_HR_LO).astype(jnp.float32)
        n = (jnp.float32(T) + (acc_n[0] - T) * in_hl
             + (acc_n[1] - T) * in_hr)
        mean = s / n
        var = (q - n * mean * mean) / (n - 1.0)
        std = jnp.sqrt(var)
        out_ref[0:DIMS, :] = jnp.where(jnp.isnan(mean), 0.0, mean)
        out_ref[DIMS:, :] = jnp.where(jnp.isnan(std), 0.0, std)


@jax.jit
def kernel(x):
    xt = x.transpose(2, 1, 0)
    tbl = jnp.asarray(np.asarray(_LIP_TILES, np.int32))
    gt = jnp.asarray(_GT_np)
    gb = jnp.asarray(_GB_np)

    lips_part = pl.pallas_call(
        _lips_kernel,
        grid_spec=pltpu.PrefetchScalarGridSpec(
            num_scalar_prefetch=1,
            grid=(N_LT, NF),
            in_specs=[
                pl.BlockSpec((DIMS, 8, FB), lambda j, f, tbl: (0, tbl[j], f)),
                pl.BlockSpec((1, 8, N_FD), lambda j, f, tbl: (j, 0, 0)),
            ],
            out_specs=pl.BlockSpec((2 * DIMS, N_FD),
                                   lambda j, f, tbl: (0, 0)),
            scratch_shapes=[
                pltpu.VMEM((DIMS, 8), jnp.float32),
                pltpu.VMEM((DIMS, 8), jnp.float32),
                pltpu.VMEM((DIMS, N_FD), jnp.float32),
                pltpu.VMEM((DIMS, N_FD), jnp.float32),
            ],
        ),
        out_shape=jax.ShapeDtypeStruct((2 * DIMS, N_FD), jnp.float32),
    )(tbl, xt, gt)

    out = pl.pallas_call(
        _tail_kernel,
        grid=(NF,),
        in_specs=[
            pl.BlockSpec((DIMS, 64, FB), lambda f: (0, 7, f)),
            pl.BlockSpec((DIMS, 64, FB), lambda f: (0, 8, f)),
            pl.BlockSpec((2 * DIMS, N_FD), lambda f: (0, 0)),
            pl.BlockSpec((128, N_FD), lambda f: (0, 0)),
        ],
        out_specs=pl.BlockSpec((2 * DIMS, N_FD), lambda f: (0, 0)),
        out_shape=jax.ShapeDtypeStruct((2 * DIMS, N_FD), jnp.float32),
        scratch_shapes=[
            pltpu.VMEM((DIMS, 128), jnp.float32),
            pltpu.VMEM((DIMS, 128), jnp.float32),
            pltpu.SMEM((2,), jnp.float32),
        ],
    )(xt, xt, lips_part, gb)

    mean_part = out[0:DIMS].T.reshape(DIMS * N_FD)
    std_part = out[DIMS:].T.reshape(DIMS * N_FD)
    return jnp.concatenate([mean_part, std_part])


# speedup vs baseline: 3.5566x; 3.5566x over previous
import jax
import jax.numpy as jnp
import numpy as np
from jax import lax
from jax.experimental import pallas as pl
from jax.experimental.pallas import tpu as pltpu

DIMS = 3
T = 8192
N_LM = 543
LIPS = ([61, 185, 40, 39, 37, 0, 267, 269, 270, 409, 291]
        + [146, 91, 181, 84, 17, 314, 405, 321, 375, 291]
        + [78, 191, 80, 81, 82, 13, 312, 311, 310, 415, 308]
        + [78, 95, 88, 178, 87, 14, 317, 402, 318, 324, 308])
HL_LO, HL_HI = 468, 489
POSE_LO, POSE_HI = 489, 522
HR_LO, HR_HI = 522, 543

_LIP_TILES = sorted({lm // 8 for lm in LIPS})
N_LT = len(_LIP_TILES)
FB = 2048
NF = T // FB

N_LIPS = len(LIPS)
_FEAT_LM = np.asarray(
    LIPS + list(range(HL_LO, HL_HI)) + list(range(POSE_LO, POSE_HI))
    + list(range(HR_LO, HR_HI)), np.int32)
N_FD = _FEAT_LM.shape[0]

_GT_np = np.zeros((N_LT, 8, N_FD), np.float32)
for _j, _lm in enumerate(LIPS):
    _GT_np[_LIP_TILES.index(_lm // 8), _lm % 8, _j] = 1.0

_GB_np = np.zeros((128, N_FD), np.float32)
for _j, _lm in enumerate(_FEAT_LM[N_LIPS:], start=N_LIPS):
    _GB_np[int(_lm) - 448, _j] = 1.0

F_HL_LO, F_HL_HI = N_LIPS, N_LIPS + 21
F_HR_LO, F_HR_HI = N_FD - 21, N_FD


def _lips_kernel(tbl_ref, lip_ref, gt_ref, out_ref, fs, fq):
    j = pl.program_id(0)

    @pl.when(j == 0)
    def _init():
        fs[...] = jnp.zeros_like(fs)
        fq[...] = jnp.zeros_like(fq)

    lb = lip_ref[...]
    gt = gt_ref[0]
    fs[...] += jnp.dot(jnp.sum(lb, axis=2), gt,
                       preferred_element_type=jnp.float32,
                       precision=jax.lax.Precision.HIGHEST)
    fq[...] += jnp.dot(jnp.sum(lb * lb, axis=2), gt,
                       preferred_element_type=jnp.float32,
                       precision=jax.lax.Precision.HIGHEST)

    @pl.when(j == N_LT - 1)
    def _emit():
        out_ref[0:DIMS, :] = fs[...]
        out_ref[DIMS:, :] = fq[...]


def _tail_kernel(ha_ref, hb_ref, lips_ref, gb_ref, out_ref,
                 acc_s, acc_q, acc_n):
    f = pl.program_id(0)

    @pl.when(f == 0)
    def _init():
        acc_s[...] = jnp.zeros_like(acc_s)
        acc_q[...] = jnp.zeros_like(acc_q)
        acc_n[0] = 0.0
        acc_n[1] = 0.0

    ba = ha_ref[...]
    bb = hb_ref[...]
    hl = ba[:, HL_LO - 448:HL_HI - 448, :]
    hr = bb[:, HR_LO - 512:HR_HI - 512, :]
    hl_bad = jnp.any(jnp.any(jnp.isnan(hl), axis=1, keepdims=True),
                     axis=0, keepdims=True).astype(jnp.float32)
    hr_bad = jnp.any(jnp.any(jnp.isnan(hr), axis=1, keepdims=True),
                     axis=0, keepdims=True).astype(jnp.float32)

    lma = lax.broadcasted_iota(jnp.int32, (1, 64, 1), 1)
    is_hl = jnp.logical_and(lma >= HL_LO - 448,
                            lma < HL_HI - 448).astype(jnp.float32)
    is_hr = jnp.logical_and(lma >= HR_LO - 512,
                            lma < HR_HI - 512).astype(jnp.float32)
    is_pad = (lma >= N_LM - 512).astype(jnp.float32)
    wa = jnp.broadcast_to(hl_bad * is_hl, (DIMS, 64, FB))
    wb = jnp.broadcast_to(hr_bad * is_hr + is_pad, (DIMS, 64, FB))
    ba2 = jnp.where(wa == 0.0, ba, 0.0)
    bb2 = jnp.where(wb == 0.0, bb, 0.0)

    acc_s[:, 0:64] += jnp.sum(ba2, axis=2)
    acc_q[:, 0:64] += jnp.sum(ba2 * ba2, axis=2)
    acc_s[:, 64:128] += jnp.sum(bb2, axis=2)
    acc_q[:, 64:128] += jnp.sum(bb2 * bb2, axis=2)
    acc_n[0] += jnp.float32(FB) - jnp.sum(hl_bad)
    acc_n[1] += jnp.float32(FB) - jnp.sum(hr_bad)

    @pl.when(f == NF - 1)
    def _finalize():
        gb = gb_ref[...]
        s = lips_ref[0:DIMS, :] + jnp.dot(
            acc_s[...], gb, preferred_element_type=jnp.float32,
            precision=jax.lax.Precision.HIGHEST)
        q = lips_ref[DIMS:, :] + jnp.dot(
            acc_q[...], gb, preferred_element_type=jnp.float32,
            precision=jax.lax.Precision.HIGHEST)
        col = lax.broadcasted_iota(jnp.int32, (DIMS, N_FD), 1)
        in_hl = jnp.logical_and(col >= F_HL_LO,
                                col < F_HL_HI).astype(jnp.float32)
        in_hr = (col >= F_HR_LO).astype(jnp.float32)
        n = (jnp.float32(T) + (acc_n[0] - T) * in_hl
             + (acc_n[1] - T) * in_hr)
        mean = s / n
        var = (q - n * mean * mean) / (n - 1.0)
        std = jnp.sqrt(var)
        out_ref[0:DIMS, :] = jnp.where(jnp.isnan(mean), 0.0, mean)
        out_ref[DIMS:, :] = jnp.where(jnp.isnan(std), 0.0, std)


@jax.jit
def kernel(x):
    xt = x.transpose(2, 1, 0)
    tbl = jnp.asarray(np.asarray(_LIP_TILES, np.int32))
    gt = jnp.asarray(_GT_np)
    gb = jnp.asarray(_GB_np)

    lips_part = pl.pallas_call(
        _lips_kernel,
        grid_spec=pltpu.PrefetchScalarGridSpec(
            num_scalar_prefetch=1,
            grid=(N_LT,),
            in_specs=[
                pl.BlockSpec((DIMS, 8, T), lambda j, tbl: (0, tbl[j], 0)),
                pl.BlockSpec((1, 8, N_FD), lambda j, tbl: (j, 0, 0)),
            ],
            out_specs=pl.BlockSpec((2 * DIMS, N_FD), lambda j, tbl: (0, 0)),
            scratch_shapes=[
                pltpu.VMEM((DIMS, N_FD), jnp.float32),
                pltpu.VMEM((DIMS, N_FD), jnp.float32),
            ],
        ),
        out_shape=jax.ShapeDtypeStruct((2 * DIMS, N_FD), jnp.float32),
    )(tbl, xt, gt)

    out = pl.pallas_call(
        _tail_kernel,
        grid=(NF,),
        in_specs=[
            pl.BlockSpec((DIMS, 64, FB), lambda f: (0, 7, f)),
            pl.BlockSpec((DIMS, 64, FB), lambda f: (0, 8, f)),
            pl.BlockSpec((2 * DIMS, N_FD), lambda f: (0, 0)),
            pl.BlockSpec((128, N_FD), lambda f: (0, 0)),
        ],
        out_specs=pl.BlockSpec((2 * DIMS, N_FD), lambda f: (0, 0)),
        out_shape=jax.ShapeDtypeStruct((2 * DIMS, N_FD), jnp.float32),
        scratch_shapes=[
            pltpu.VMEM((DIMS, 128), jnp.float32),
            pltpu.VMEM((DIMS, 128), jnp.float32),
            pltpu.SMEM((2,), jnp.float32),
        ],
    )(xt, xt, lips_part, gb)

    mean_part = out[0:DIMS].T.reshape(DIMS * N_FD)
    std_part = out[DIMS:].T.reshape(DIMS * N_FD)
    return jnp.concatenate([mean_part, std_part])


# speedup vs baseline: 3.6251x; 1.0193x over previous
import jax
import jax.numpy as jnp
import numpy as np
from jax import lax
from jax.experimental import pallas as pl
from jax.experimental.pallas import tpu as pltpu

DIMS = 3
T = 8192
N_LM = 543
LIPS = ([61, 185, 40, 39, 37, 0, 267, 269, 270, 409, 291]
        + [146, 91, 181, 84, 17, 314, 405, 321, 375, 291]
        + [78, 191, 80, 81, 82, 13, 312, 311, 310, 415, 308]
        + [78, 95, 88, 178, 87, 14, 317, 402, 318, 324, 308])
HL_LO, HL_HI = 468, 489
POSE_LO, POSE_HI = 489, 522
HR_LO, HR_HI = 522, 543

_LIP_TILES = sorted({lm // 8 for lm in LIPS})
N_LT = len(_LIP_TILES)

N_LIPS = len(LIPS)
_FEAT_LM = np.asarray(
    LIPS + list(range(HL_LO, HL_HI)) + list(range(POSE_LO, POSE_HI))
    + list(range(HR_LO, HR_HI)), np.int32)
N_FD = _FEAT_LM.shape[0]

_GT_np = np.zeros((N_LT, 8, N_FD), np.float32)
for _j, _lm in enumerate(LIPS):
    _GT_np[_LIP_TILES.index(_lm // 8), _lm % 8, _j] = 1.0

_GB_np = np.zeros((128, N_FD), np.float32)
for _j, _lm in enumerate(_FEAT_LM[N_LIPS:], start=N_LIPS):
    _GB_np[int(_lm) - 448, _j] = 1.0

F_HL_LO, F_HL_HI = N_LIPS, N_LIPS + 21
F_HR_LO, F_HR_HI = N_FD - 21, N_FD

HP = jax.lax.Precision.HIGHEST


def _stats_kernel(tbl_ref, lip_ref, ha_ref, hb_ref, gt_ref, gb_ref,
                  out_ref, fs, fq, acc_n):
    j = pl.program_id(0)

    @pl.when(j == 0)
    def _tail():
        ba = ha_ref[...]
        bb = hb_ref[...]
        hl = ba[:, HL_LO - 448:HL_HI - 448, :]
        hr = bb[:, HR_LO - 512:HR_HI - 512, :]
        hl_bad = jnp.any(jnp.any(jnp.isnan(hl), axis=1, keepdims=True),
                         axis=0, keepdims=True).astype(jnp.float32)
        hr_bad = jnp.any(jnp.any(jnp.isnan(hr), axis=1, keepdims=True),
                         axis=0, keepdims=True).astype(jnp.float32)

        lma = lax.broadcasted_iota(jnp.int32, (1, 64, 1), 1)
        is_hl = jnp.logical_and(lma >= HL_LO - 448,
                                lma < HL_HI - 448).astype(jnp.float32)
        is_hr = jnp.logical_and(lma >= HR_LO - 512,
                                lma < HR_HI - 512).astype(jnp.float32)
        is_pad = (lma >= N_LM - 512).astype(jnp.float32)
        wa = jnp.broadcast_to(hl_bad * is_hl, (DIMS, 64, T))
        wb = jnp.broadcast_to(hr_bad * is_hr + is_pad, (DIMS, 64, T))
        ba2 = jnp.where(wa == 0.0, ba, 0.0)
        bb2 = jnp.where(wb == 0.0, bb, 0.0)

        gb = gb_ref[...]
        ts = jnp.concatenate([jnp.sum(ba2, axis=2),
                              jnp.sum(bb2, axis=2)], axis=1)
        tq = jnp.concatenate([jnp.sum(ba2 * ba2, axis=2),
                              jnp.sum(bb2 * bb2, axis=2)], axis=1)
        fs[...] = jnp.dot(ts, gb, preferred_element_type=jnp.float32,
                          precision=HP)
        fq[...] = jnp.dot(tq, gb, preferred_element_type=jnp.float32,
                          precision=HP)
        acc_n[0] = jnp.float32(T) - jnp.sum(hl_bad)
        acc_n[1] = jnp.float32(T) - jnp.sum(hr_bad)

    lb = lip_ref[...]
    gt = gt_ref[0]
    fs[...] += jnp.dot(jnp.sum(lb, axis=2), gt,
                       preferred_element_type=jnp.float32, precision=HP)
    fq[...] += jnp.dot(jnp.sum(lb * lb, axis=2), gt,
                       preferred_element_type=jnp.float32, precision=HP)

    @pl.when(j == N_LT - 1)
    def _finalize():
        s = fs[...]
        q = fq[...]
        col = lax.broadcasted_iota(jnp.int32, (DIMS, N_FD), 1)
        in_hl = jnp.logical_and(col >= F_HL_LO,
                                col < F_HL_HI).astype(jnp.float32)
        in_hr = (col >= F_HR_LO).astype(jnp.float32)
        n = (jnp.float32(T) + (acc_n[0] - T) * in_hl
             + (acc_n[1] - T) * in_hr)
        mean = s / n
        var = (q - n * mean * mean) / (n - 1.0)
        std = jnp.sqrt(var)
        out_ref[0:DIMS, :] = jnp.where(jnp.isnan(mean), 0.0, mean)
        out_ref[DIMS:, :] = jnp.where(jnp.isnan(std), 0.0, std)


@jax.jit
def kernel(x):
    xt = x.transpose(2, 1, 0)
    tbl = jnp.asarray(np.asarray(_LIP_TILES, np.int32))
    gt = jnp.asarray(_GT_np)
    gb = jnp.asarray(_GB_np)

    out = pl.pallas_call(
        _stats_kernel,
        grid_spec=pltpu.PrefetchScalarGridSpec(
            num_scalar_prefetch=1,
            grid=(N_LT,),
            in_specs=[
                pl.BlockSpec((DIMS, 8, T), lambda j, tbl: (0, tbl[j], 0)),
                pl.BlockSpec((DIMS, 64, T), lambda j, tbl: (0, 7, 0)),
                pl.BlockSpec((DIMS, 64, T), lambda j, tbl: (0, 8, 0)),
                pl.BlockSpec((1, 8, N_FD), lambda j, tbl: (j, 0, 0)),
                pl.BlockSpec((128, N_FD), lambda j, tbl: (0, 0)),
            ],
            out_specs=pl.BlockSpec((2 * DIMS, N_FD), lambda j, tbl: (0, 0)),
            scratch_shapes=[
                pltpu.VMEM((DIMS, N_FD), jnp.float32),
                pltpu.VMEM((DIMS, N_FD), jnp.float32),
                pltpu.SMEM((2,), jnp.float32),
            ],
        ),
        out_shape=jax.ShapeDtypeStruct((2 * DIMS, N_FD), jnp.float32),
    )(tbl, xt, xt, xt, gt, gb)

    mean_part = out[0:DIMS].T.reshape(DIMS * N_FD)
    std_part = out[DIMS:].T.reshape(DIMS * N_FD)
    return jnp.concatenate([mean_part, std_part])


# speedup vs baseline: 3.9204x; 1.0814x over previous
import jax
import jax.numpy as jnp
import numpy as np
from jax import lax
from jax.experimental import pallas as pl
from jax.experimental.pallas import tpu as pltpu

DIMS = 3
T = 8192
N_LM = 543
LIPS = ([61, 185, 40, 39, 37, 0, 267, 269, 270, 409, 291]
        + [146, 91, 181, 84, 17, 314, 405, 321, 375, 291]
        + [78, 191, 80, 81, 82, 13, 312, 311, 310, 415, 308]
        + [78, 95, 88, 178, 87, 14, 317, 402, 318, 324, 308])
HL_LO, HL_HI = 468, 489
POSE_LO, POSE_HI = 489, 522
HR_LO, HR_HI = 522, 543

LT_W = 16
_LIP_TILES = sorted({lm // LT_W for lm in LIPS})
N_LT = len(_LIP_TILES)

N_LIPS = len(LIPS)
_FEAT_LM = np.asarray(
    LIPS + list(range(HL_LO, HL_HI)) + list(range(POSE_LO, POSE_HI))
    + list(range(HR_LO, HR_HI)), np.int32)
N_FD = _FEAT_LM.shape[0]

_GT_np = np.zeros((N_LT, LT_W, N_FD), np.float32)
for _j, _lm in enumerate(LIPS):
    _GT_np[_LIP_TILES.index(_lm // LT_W), _lm % LT_W, _j] = 1.0

_GB_np = np.zeros((128, N_FD), np.float32)
for _j, _lm in enumerate(_FEAT_LM[N_LIPS:], start=N_LIPS):
    _GB_np[int(_lm) - 448, _j] = 1.0

F_HL_LO, F_HL_HI = N_LIPS, N_LIPS + 21
F_HR_LO, F_HR_HI = N_FD - 21, N_FD

HP = jax.lax.Precision.HIGHEST


def _stats_kernel(tbl_ref, lip_ref, ha_ref, hb_ref, gt_ref, gb_ref,
                  out_ref, fs, fq, acc_n):
    j = pl.program_id(0)

    @pl.when(j == 0)
    def _tail():
        ba = ha_ref[...]
        bb = hb_ref[...]
        hl = ba[:, HL_LO - 448:HL_HI - 448, :]
        hr = bb[:, HR_LO - 512:HR_HI - 512, :]
        hl_bad = jnp.any(jnp.any(jnp.isnan(hl), axis=1, keepdims=True),
                         axis=0, keepdims=True).astype(jnp.float32)
        hr_bad = jnp.any(jnp.any(jnp.isnan(hr), axis=1, keepdims=True),
                         axis=0, keepdims=True).astype(jnp.float32)

        lma = lax.broadcasted_iota(jnp.int32, (1, 64, 1), 1)
        is_hl = jnp.logical_and(lma >= HL_LO - 448,
                                lma < HL_HI - 448).astype(jnp.float32)
        is_hr = jnp.logical_and(lma >= HR_LO - 512,
                                lma < HR_HI - 512).astype(jnp.float32)
        is_pad = (lma >= N_LM - 512).astype(jnp.float32)
        wa = jnp.broadcast_to(hl_bad * is_hl, (DIMS, 64, T))
        wb = jnp.broadcast_to(hr_bad * is_hr + is_pad, (DIMS, 64, T))
        ba2 = jnp.where(wa == 0.0, ba, 0.0)
        bb2 = jnp.where(wb == 0.0, bb, 0.0)

        gb = gb_ref[...]
        ts = jnp.concatenate([jnp.sum(ba2, axis=2),
                              jnp.sum(bb2, axis=2)], axis=1)
        tq = jnp.concatenate([jnp.sum(ba2 * ba2, axis=2),
                              jnp.sum(bb2 * bb2, axis=2)], axis=1)
        fs[...] = jnp.dot(ts, gb, preferred_element_type=jnp.float32,
                          precision=HP)
        fq[...] = jnp.dot(tq, gb, preferred_element_type=jnp.float32,
                          precision=HP)
        acc_n[0] = jnp.float32(T) - jnp.sum(hl_bad)
        acc_n[1] = jnp.float32(T) - jnp.sum(hr_bad)

    lb = lip_ref[...]
    gt = gt_ref[0]
    fs[...] += jnp.dot(jnp.sum(lb, axis=2), gt,
                       preferred_element_type=jnp.float32, precision=HP)
    fq[...] += jnp.dot(jnp.sum(lb * lb, axis=2), gt,
                       preferred_element_type=jnp.float32, precision=HP)

    @pl.when(j == N_LT - 1)
    def _finalize():
        s = fs[...]
        q = fq[...]
        col = lax.broadcasted_iota(jnp.int32, (DIMS, N_FD), 1)
        in_hl = jnp.logical_and(col >= F_HL_LO,
                                col < F_HL_HI).astype(jnp.float32)
        in_hr = (col >= F_HR_LO).astype(jnp.float32)
        n = (jnp.float32(T) + (acc_n[0] - T) * in_hl
             + (acc_n[1] - T) * in_hr)
        mean = s / n
        var = (q - n * mean * mean) / (n - 1.0)
        std = jnp.sqrt(var)
        out_ref[0:DIMS, :] = jnp.where(jnp.isnan(mean), 0.0, mean)
        out_ref[DIMS:, :] = jnp.where(jnp.isnan(std), 0.0, std)


@jax.jit
def kernel(x):
    xt = x.transpose(2, 1, 0)
    tbl = jnp.asarray(np.asarray(_LIP_TILES, np.int32))
    gt = jnp.asarray(_GT_np)
    gb = jnp.asarray(_GB_np)

    out = pl.pallas_call(
        _stats_kernel,
        grid_spec=pltpu.PrefetchScalarGridSpec(
            num_scalar_prefetch=1,
            grid=(N_LT,),
            in_specs=[
                pl.BlockSpec((DIMS, LT_W, T), lambda j, tbl: (0, tbl[j], 0)),
                pl.BlockSpec((DIMS, 64, T), lambda j, tbl: (0, 7, 0)),
                pl.BlockSpec((DIMS, 64, T), lambda j, tbl: (0, 8, 0)),
                pl.BlockSpec((1, LT_W, N_FD), lambda j, tbl: (j, 0, 0)),
                pl.BlockSpec((128, N_FD), lambda j, tbl: (0, 0)),
            ],
            out_specs=pl.BlockSpec((2 * DIMS, N_FD), lambda j, tbl: (0, 0)),
            scratch_shapes=[
                pltpu.VMEM((DIMS, N_FD), jnp.float32),
                pltpu.VMEM((DIMS, N_FD), jnp.float32),
                pltpu.SMEM((2,), jnp.float32),
            ],
        ),
        out_shape=jax.ShapeDtypeStruct((2 * DIMS, N_FD), jnp.float32),
    )(tbl, xt, xt, xt, gt, gb)

    mean_part = out[0:DIMS].T.reshape(DIMS * N_FD)
    std_part = out[DIMS:].T.reshape(DIMS * N_FD)
    return jnp.concatenate([mean_part, std_part])


# speedup vs baseline: 3.9615x; 1.0105x over previous
import jax
import jax.numpy as jnp
import numpy as np
from jax import lax
from jax.experimental import pallas as pl
from jax.experimental.pallas import tpu as pltpu

DIMS = 3
T = 8192
N_LM = 543
LIPS = ([61, 185, 40, 39, 37, 0, 267, 269, 270, 409, 291]
        + [146, 91, 181, 84, 17, 314, 405, 321, 375, 291]
        + [78, 191, 80, 81, 82, 13, 312, 311, 310, 415, 308]
        + [78, 95, 88, 178, 87, 14, 317, 402, 318, 324, 308])
HL_LO, HL_HI = 468, 489
POSE_LO, POSE_HI = 489, 522
HR_LO, HR_HI = 522, 543

LT_W = 16
_LIP_TILES = sorted({lm // LT_W for lm in LIPS})
N_LT = len(_LIP_TILES)

N_LIPS = len(LIPS)
_FEAT_LM = np.asarray(
    LIPS + list(range(HL_LO, HL_HI)) + list(range(POSE_LO, POSE_HI))
    + list(range(HR_LO, HR_HI)), np.int32)
N_FD = _FEAT_LM.shape[0]

_GT_np = np.zeros((N_LT, LT_W, N_FD), np.float32)
for _j, _lm in enumerate(LIPS):
    _GT_np[_LIP_TILES.index(_lm // LT_W), _lm % LT_W, _j] = 1.0

_GB_np = np.zeros((128, N_FD), np.float32)
for _j, _lm in enumerate(_FEAT_LM[N_LIPS:], start=N_LIPS):
    _GB_np[int(_lm) - 448, _j] = 1.0

F_HL_LO, F_HL_HI = N_LIPS, N_LIPS + 21
F_HR_LO, F_HR_HI = N_FD - 21, N_FD

HP = jax.lax.Precision.HIGHEST


def _stats_kernel(tbl_ref, lip_ref, ha_ref, hb_ref, gt_ref, gb_ref,
                  out_ref, fs, fq, acc_n):
    j = pl.program_id(0)

    @pl.when(j == 0)
    def _tail():
        ba = ha_ref[...]
        bb = hb_ref[...]
        hl = ba[:, HL_LO - 448:HL_HI - 448, :]
        hr = bb[:, HR_LO - 512:HR_HI - 512, :]
        hl_bad = jnp.any(jnp.any(jnp.isnan(hl), axis=1, keepdims=True),
                         axis=0, keepdims=True).astype(jnp.float32)
        hr_bad = jnp.any(jnp.any(jnp.isnan(hr), axis=1, keepdims=True),
                         axis=0, keepdims=True).astype(jnp.float32)

        nh = HL_HI - HL_LO
        wl = jnp.broadcast_to(hl_bad, (DIMS, nh, T)) == 0.0
        wr = jnp.broadcast_to(hr_bad, (DIMS, nh, T)) == 0.0
        hl2 = jnp.where(wl, hl, 0.0)
        hr2 = jnp.where(wr, hr, 0.0)

        s_a = jnp.sum(ba, axis=2)
        q_a = jnp.sum(ba * ba, axis=2)
        s_b = jnp.sum(bb, axis=2)
        q_b = jnp.sum(bb * bb, axis=2)
        z = jnp.zeros((DIMS, 64 - (HR_HI - 512)), jnp.float32)

        gb = gb_ref[...]
        ts = jnp.concatenate([
            s_a[:, :HL_LO - 448], jnp.sum(hl2, axis=2), s_a[:, HL_HI - 448:],
            s_b[:, :HR_LO - 512], jnp.sum(hr2, axis=2), z], axis=1)
        tq = jnp.concatenate([
            q_a[:, :HL_LO - 448], jnp.sum(hl2 * hl2, axis=2),
            q_a[:, HL_HI - 448:],
            q_b[:, :HR_LO - 512], jnp.sum(hr2 * hr2, axis=2), z], axis=1)
        fs[...] = jnp.dot(ts, gb, preferred_element_type=jnp.float32,
                          precision=HP)
        fq[...] = jnp.dot(tq, gb, preferred_element_type=jnp.float32,
                          precision=HP)
        acc_n[0] = jnp.float32(T) - jnp.sum(hl_bad)
        acc_n[1] = jnp.float32(T) - jnp.sum(hr_bad)

    lb = lip_ref[...]
    gt = gt_ref[0]
    fs[...] += jnp.dot(jnp.sum(lb, axis=2), gt,
                       preferred_element_type=jnp.float32, precision=HP)
    fq[...] += jnp.dot(jnp.sum(lb * lb, axis=2), gt,
                       preferred_element_type=jnp.float32, precision=HP)

    @pl.when(j == N_LT - 1)
    def _finalize():
        s = fs[...]
        q = fq[...]
        col = lax.broadcasted_iota(jnp.int32, (DIMS, N_FD), 1)
        in_hl = jnp.logical_and(col >= F_HL_LO,
                                col < F_HL_HI).astype(jnp.float32)
        in_hr = (col >= F_HR_LO).astype(jnp.float32)
        n = (jnp.float32(T) + (acc_n[0] - T) * in_hl
             + (acc_n[1] - T) * in_hr)
        mean = s / n
        var = (q - n * mean * mean) / (n - 1.0)
        std = jnp.sqrt(var)
        out_ref[0:DIMS, :] = jnp.where(jnp.isnan(mean), 0.0, mean)
        out_ref[DIMS:, :] = jnp.where(jnp.isnan(std), 0.0, std)


@jax.jit
def kernel(x):
    xt = x.transpose(2, 1, 0)
    tbl = jnp.asarray(np.asarray(_LIP_TILES, np.int32))
    gt = jnp.asarray(_GT_np)
    gb = jnp.asarray(_GB_np)

    out = pl.pallas_call(
        _stats_kernel,
        grid_spec=pltpu.PrefetchScalarGridSpec(
            num_scalar_prefetch=1,
            grid=(N_LT,),
            in_specs=[
                pl.BlockSpec((DIMS, LT_W, T), lambda j, tbl: (0, tbl[j], 0)),
                pl.BlockSpec((DIMS, 64, T), lambda j, tbl: (0, 7, 0)),
                pl.BlockSpec((DIMS, 64, T), lambda j, tbl: (0, 8, 0)),
                pl.BlockSpec((1, LT_W, N_FD), lambda j, tbl: (j, 0, 0)),
                pl.BlockSpec((128, N_FD), lambda j, tbl: (0, 0)),
            ],
            out_specs=pl.BlockSpec((2 * DIMS, N_FD), lambda j, tbl: (0, 0)),
            scratch_shapes=[
                pltpu.VMEM((DIMS, N_FD), jnp.float32),
                pltpu.VMEM((DIMS, N_FD), jnp.float32),
                pltpu.SMEM((2,), jnp.float32),
            ],
        ),
        out_shape=jax.ShapeDtypeStruct((2 * DIMS, N_FD), jnp.float32),
    )(tbl, xt, xt, xt, gt, gb)

    mean_part = out[0:DIMS].T.reshape(DIMS * N_FD)
    std_part = out[DIMS:].T.reshape(DIMS * N_FD)
    return jnp.concatenate([mean_part, std_part])
